# baseline (speedup 1.0000x reference)
import math
from functools import partial

import jax
import jax.numpy as jnp
from jax.experimental import pallas as pl
from jax.experimental.pallas import tpu as pltpu


def _round_up(v, m):
    return ((v + m - 1) // m) * m


def _prep_body(adj_ref, asl_ref, dinv_ref, *, tile):
    i = pl.program_id(0)
    a = adj_ref[...]
    deg = jnp.sum(a, axis=1, keepdims=True) + 2.0
    dinv_ref[...] = jax.lax.rsqrt(deg)
    r = jax.lax.broadcasted_iota(jnp.int32, a.shape, 0) + i * tile
    c = jax.lax.broadcasted_iota(jnp.int32, a.shape, 1)
    asl_ref[...] = jnp.where(r == c, 1, a.astype(jnp.int8))


def _gcn_body(*refs, relu, has_u, has_score, has_y, softmax_classes, tile):
    it = iter(refs)
    x_ref = next(it)
    adj_ref = next(it)
    dinv_ref = next(it)
    w_ref = next(it)
    b_ref = next(it)
    u_ref = next(it) if has_u else None
    p_ref = next(it) if has_score else None
    w2_ref = next(it) if has_y else None
    o_ref = next(it)
    s_ref = next(it) if has_score else None
    y_ref = next(it) if has_y else None

    i = pl.program_id(0)
    dinv = dinv_ref[...]
    w = w_ref[...]
    xw = jnp.dot(x_ref[...].astype(jnp.bfloat16), w,
                 preferred_element_type=jnp.float32)
    if has_u:
        xw = xw + u_ref[...]
    xws_bf = (xw * dinv).astype(jnp.bfloat16)

    adj_t = adj_ref[...].astype(jnp.bfloat16)
    x_t = x_ref[pl.ds(i * tile, tile), :]
    xw_t = jnp.dot(x_t.astype(jnp.bfloat16), w,
                   preferred_element_type=jnp.float32)
    if has_u:
        xw_t = xw_t + u_ref[pl.ds(i * tile, tile), :]
    dinv_t = dinv_ref[pl.ds(i * tile, tile), :]
    xws_bf_t = (xw_t * dinv_t).astype(jnp.bfloat16)

    prop = jnp.dot(adj_t, xws_bf, preferred_element_type=jnp.float32)
    prop = prop - xws_bf_t.astype(jnp.float32)
    out = (prop + 2.0 * xw_t * dinv_t) * dinv_t + b_ref[...]
    if relu:
        out = jnp.maximum(out, 0.0)

    if softmax_classes is not None:
        col = jax.lax.broadcasted_iota(jnp.int32, out.shape, 1)
        logits = jnp.where(col < softmax_classes, out, -jnp.inf)
        m = jnp.max(logits, axis=-1, keepdims=True)
        s = logits - m
        lse = jnp.log(jnp.sum(jnp.exp(s), axis=-1, keepdims=True))
        out = s - lse

    o_ref[...] = out.astype(o_ref.dtype)

    if has_score:
        raw = jnp.sum(out * p_ref[...], axis=-1, keepdims=True)
        s_ref[...] = jnp.tanh(raw)

    if has_y:
        y_ref[...] = jnp.dot(out.astype(jnp.bfloat16), w2_ref[...],
                             preferred_element_type=jnp.float32)


def _bbt_body(brow_ref, ball_ref, gcol_ref, grow_ref,
              asl_ref, dinv_ref, *, tile):
    i = pl.program_id(0)
    acc = jax.lax.dot_general(
        brow_ref[...], ball_ref[...],
        (((1,), (1,)), ((), ())),
        preferred_element_type=jnp.int32)
    r = jax.lax.broadcasted_iota(jnp.int32, acc.shape, 0) + i * tile
    c = jax.lax.broadcasted_iota(jnp.int32, acc.shape, 1)
    diag = r == c
    keep = (gcol_ref[...] > 0) & (grow_ref[...] > 0) & ~diag
    out = jnp.where(keep, acc, 0)
    asl_ref[...] = jnp.where(diag, 1, out).astype(jnp.int8)
    deg = jnp.sum(out, axis=1, keepdims=True).astype(jnp.float32) + 2.0
    dinv_ref[...] = jax.lax.rsqrt(deg)


_PARALLEL = pltpu.CompilerParams(dimension_semantics=("parallel",))


def _prep(adj, n_tiles=4):
    N = adj.shape[0]
    tile = N // n_tiles
    return pl.pallas_call(
        partial(_prep_body, tile=tile),
        out_shape=(jax.ShapeDtypeStruct((N, N), jnp.int8),
                   jax.ShapeDtypeStruct((N, 1), jnp.float32)),
        grid=(n_tiles,),
        in_specs=[pl.BlockSpec((tile, N), lambda i: (i, 0))],
        out_specs=(pl.BlockSpec((tile, N), lambda i: (i, 0)),
                   pl.BlockSpec((tile, 1), lambda i: (i, 0))),
        compiler_params=_PARALLEL,
    )(adj)


def _gcn_conv(x, adj_bf, dinv, w, b, *, relu, u=None, p_unit=None, w2=None,
              softmax_classes=None, n_tiles=1):
    M = adj_bf.shape[0]
    Cin = x.shape[1]
    Cout = w.shape[1]
    tile = M // n_tiles
    b2 = b.reshape(1, Cout).astype(jnp.float32)
    has_u = u is not None
    has_score = p_unit is not None
    has_y = w2 is not None

    inputs = [x.astype(jnp.float32), adj_bf, dinv,
              w.astype(jnp.bfloat16), b2]
    in_specs = [
        pl.BlockSpec((M, Cin), lambda i: (0, 0)),
        pl.BlockSpec((tile, M), lambda i: (i, 0)),
        pl.BlockSpec((M, 1), lambda i: (0, 0)),
        pl.BlockSpec((Cin, Cout), lambda i: (0, 0)),
        pl.BlockSpec((1, Cout), lambda i: (0, 0)),
    ]
    if has_u:
        inputs.append(u)
        in_specs.append(pl.BlockSpec((M, Cout), lambda i: (0, 0)))
    if has_score:
        inputs.append(p_unit.reshape(1, Cout).astype(jnp.float32))
        in_specs.append(pl.BlockSpec((1, Cout), lambda i: (0, 0)))
    if has_y:
        inputs.append(w2.astype(jnp.bfloat16))
        in_specs.append(
            pl.BlockSpec((w2.shape[0], w2.shape[1]), lambda i: (0, 0)))

    out_shape = [jax.ShapeDtypeStruct((M, Cout), jnp.float32)]
    out_specs = [pl.BlockSpec((tile, Cout), lambda i: (i, 0))]
    if has_score:
        out_shape.append(jax.ShapeDtypeStruct((M, 1), jnp.float32))
        out_specs.append(pl.BlockSpec((tile, 1), lambda i: (i, 0)))
    if has_y:
        C2 = w2.shape[1]
        out_shape.append(jax.ShapeDtypeStruct((M, C2), jnp.float32))
        out_specs.append(pl.BlockSpec((tile, C2), lambda i: (i, 0)))

    res = pl.pallas_call(
        partial(_gcn_body, relu=relu, has_u=has_u, has_score=has_score,
                has_y=has_y, softmax_classes=softmax_classes, tile=tile),
        out_shape=tuple(out_shape),
        grid=(n_tiles,),
        in_specs=in_specs,
        out_specs=tuple(out_specs),
        compiler_params=_PARALLEL,
    )(*inputs)
    return res if len(out_shape) > 1 else res[0]


def _bbt(bmat, gate, n_tiles):
    Mp, K = bmat.shape
    tile = Mp // n_tiles
    gcol = gate.reshape(Mp, 1)
    grow = gate.reshape(1, Mp)
    return pl.pallas_call(
        partial(_bbt_body, tile=tile),
        out_shape=(jax.ShapeDtypeStruct((Mp, Mp), jnp.int8),
                   jax.ShapeDtypeStruct((Mp, 1), jnp.float32)),
        grid=(n_tiles,),
        in_specs=[pl.BlockSpec((tile, K), lambda i: (i, 0)),
                  pl.BlockSpec((Mp, K), lambda i: (0, 0)),
                  pl.BlockSpec((tile, 1), lambda i: (i, 0)),
                  pl.BlockSpec((1, Mp), lambda i: (0, 0))],
        out_specs=(pl.BlockSpec((tile, Mp), lambda i: (i, 0)),
                   pl.BlockSpec((tile, 1), lambda i: (i, 0))),
        compiler_params=_PARALLEL,
    )(bmat, bmat, gcol, grow)


def _up_scatter(y, perm, k, rows):
    return jnp.zeros((rows, y.shape[1]), y.dtype).at[perm[:k]].set(y[:k])


def kernel(x_feat, pos, adj,
           down_w_0, down_w_1, down_w_2,
           down_b_0, down_b_1, down_b_2,
           pool_p_0, pool_p_1,
           up_w_0, up_w_1, up_b_0, up_b_1):
    N = adj.shape[0]
    num_classes = up_w_1.shape[1]
    c1 = up_w_0.shape[1]
    c0 = down_w_0.shape[1]

    p0_unit = pool_p_0 / jnp.linalg.norm(pool_p_0)
    p1_unit = pool_p_1 / jnp.linalg.norm(pool_p_1)

    w_up0_top = up_w_0[:c1]
    w_up0_bot = up_w_0[c1:]
    w_fin_pad = jnp.zeros((up_w_1.shape[0], 128),
                          jnp.float32).at[:, :num_classes].set(up_w_1)
    b_fin_pad = jnp.zeros((128,), jnp.float32).at[:num_classes].set(up_b_1)
    w_fin_top = w_fin_pad[:c0]
    w_fin_bot = w_fin_pad[c0:]

    asl0, dinv0 = _prep(adj.astype(jnp.float32))

    x0 = jnp.concatenate([x_feat, pos], axis=-1).astype(jnp.float32)
    x0out, score0 = _gcn_conv(x0, asl0, dinv0, down_w_0, down_b_0,
                              relu=True, p_unit=p0_unit)

    n0 = N
    k1 = int(math.ceil(0.8 * n0))
    kpad1 = min(_round_up(k1, 8), N)
    Mp1 = _round_up(kpad1, 128)
    perm1 = jnp.argsort(-score0[:, 0])[:kpad1]
    perm1 = jnp.concatenate(
        [perm1, jnp.zeros((Mp1 - kpad1,), perm1.dtype)])
    gate1 = (jnp.arange(Mp1) < k1).astype(jnp.float32)

    sc1 = score0[perm1, 0] * gate1
    x1 = x0out[perm1] * sc1[:, None]
    b1 = asl0[perm1]

    adj1, dinv1 = _bbt(b1, gate1.astype(jnp.int32), n_tiles=2)

    x1out, score1 = _gcn_conv(x1, adj1, dinv1, down_w_1, down_b_1,
                              relu=True, p_unit=p1_unit)

    n1 = k1
    k2 = int(math.ceil(0.8 * n1))
    kpad2 = min(_round_up(k2, 8), kpad1)
    Mp2 = _round_up(kpad2, 128)
    valid = jnp.arange(Mp1) < n1
    masked = jnp.where(valid, score1[:, 0], -jnp.inf)
    perm2 = jnp.argsort(-masked)[:kpad2]
    perm2 = jnp.concatenate(
        [perm2, jnp.zeros((Mp2 - kpad2,), perm2.dtype)])
    gate2 = (jnp.arange(Mp2) < k2).astype(jnp.float32)

    sc2 = score1[perm2, 0] * gate2
    x2 = x1out[perm2] * sc2[:, None]
    b2 = adj1[perm2]

    adj2, dinv2 = _bbt(b2, gate2.astype(jnp.int32), n_tiles=2)

    x2out, y2 = _gcn_conv(x2, adj2, dinv2, down_w_2, down_b_2,
                          relu=True, w2=w_up0_bot)

    u1 = _up_scatter(y2, perm2, k2, Mp1)
    xu1, y1 = _gcn_conv(x1out, adj1, dinv1, w_up0_top, up_b_0,
                        relu=True, u=u1, w2=w_fin_bot)

    u0 = _up_scatter(y1, perm1, k1, N)
    out = _gcn_conv(x0out, asl0, dinv0, w_fin_top, b_fin_pad, relu=False,
                    u=u0, softmax_classes=num_classes)
    return out[:, :num_classes]


# speedup vs baseline: 1.0082x; 1.0082x over previous
import math
from functools import partial

import jax
import jax.numpy as jnp
from jax.experimental import pallas as pl
from jax.experimental.pallas import tpu as pltpu


def _round_up(v, m):
    return ((v + m - 1) // m) * m


def _prep_body(adj_ref, asl_ref, dinv_ref, *, tile):
    i = pl.program_id(0)
    a = adj_ref[...]
    deg = jnp.sum(a, axis=1, keepdims=True) + 2.0
    dinv_ref[...] = jax.lax.rsqrt(deg)
    r = jax.lax.broadcasted_iota(jnp.int32, a.shape, 0) + i * tile
    c = jax.lax.broadcasted_iota(jnp.int32, a.shape, 1)
    asl_ref[...] = jnp.where(r == c, 1, a.astype(jnp.int8))


def _gcn_body(*refs, relu, has_u, has_score, has_y, softmax_classes, tile):
    it = iter(refs)
    x_ref = next(it)
    adj_ref = next(it)
    dinv_ref = next(it)
    w_ref = next(it)
    b_ref = next(it)
    u_ref = next(it) if has_u else None
    p_ref = next(it) if has_score else None
    w2_ref = next(it) if has_y else None
    o_ref = next(it)
    s_ref = next(it) if has_score else None
    y_ref = next(it) if has_y else None

    i = pl.program_id(0)
    dinv = dinv_ref[...]
    w = w_ref[...]
    xw = jnp.dot(x_ref[...].astype(jnp.bfloat16), w,
                 preferred_element_type=jnp.float32)
    if has_u:
        xw = xw + u_ref[...]
    xws_bf = (xw * dinv).astype(jnp.bfloat16)

    adj_t = adj_ref[...].astype(jnp.bfloat16)
    x_t = x_ref[pl.ds(i * tile, tile), :]
    xw_t = jnp.dot(x_t.astype(jnp.bfloat16), w,
                   preferred_element_type=jnp.float32)
    if has_u:
        xw_t = xw_t + u_ref[pl.ds(i * tile, tile), :]
    dinv_t = dinv_ref[pl.ds(i * tile, tile), :]
    xws_bf_t = (xw_t * dinv_t).astype(jnp.bfloat16)

    prop = jnp.dot(adj_t, xws_bf, preferred_element_type=jnp.float32)
    prop = prop - xws_bf_t.astype(jnp.float32)
    out = (prop + 2.0 * xw_t * dinv_t) * dinv_t + b_ref[...]
    if relu:
        out = jnp.maximum(out, 0.0)

    if softmax_classes is not None:
        col = jax.lax.broadcasted_iota(jnp.int32, out.shape, 1)
        logits = jnp.where(col < softmax_classes, out, -jnp.inf)
        m = jnp.max(logits, axis=-1, keepdims=True)
        s = logits - m
        lse = jnp.log(jnp.sum(jnp.exp(s), axis=-1, keepdims=True))
        out = s - lse

    o_ref[...] = out.astype(o_ref.dtype)

    if has_score:
        raw = jnp.sum(out * p_ref[...], axis=-1, keepdims=True)
        s_ref[...] = jnp.tanh(raw)

    if has_y:
        y_ref[...] = jnp.dot(out.astype(jnp.bfloat16), w2_ref[...],
                             preferred_element_type=jnp.float32)


def _bbt_body(brow_ref, ball_ref, gcol_ref, grow_ref,
              asl_ref, dinv_ref, *, tile):
    i = pl.program_id(0)
    acc = jax.lax.dot_general(
        brow_ref[...], ball_ref[...],
        (((1,), (1,)), ((), ())),
        preferred_element_type=jnp.int32)
    r = jax.lax.broadcasted_iota(jnp.int32, acc.shape, 0) + i * tile
    c = jax.lax.broadcasted_iota(jnp.int32, acc.shape, 1)
    diag = r == c
    keep = (gcol_ref[...] > 0) & (grow_ref[...] > 0) & ~diag
    out = jnp.where(keep, acc, 0)
    asl_ref[...] = jnp.where(diag, 1, out).astype(jnp.int8)
    deg = jnp.sum(out, axis=1, keepdims=True).astype(jnp.float32) + 2.0
    dinv_ref[...] = jax.lax.rsqrt(deg)


_PARALLEL = pltpu.CompilerParams(dimension_semantics=("parallel",))


def _prep(adj, n_tiles=4):
    N = adj.shape[0]
    tile = N // n_tiles
    return pl.pallas_call(
        partial(_prep_body, tile=tile),
        out_shape=(jax.ShapeDtypeStruct((N, N), jnp.int8),
                   jax.ShapeDtypeStruct((N, 1), jnp.float32)),
        grid=(n_tiles,),
        in_specs=[pl.BlockSpec((tile, N), lambda i: (i, 0))],
        out_specs=(pl.BlockSpec((tile, N), lambda i: (i, 0)),
                   pl.BlockSpec((tile, 1), lambda i: (i, 0))),
        compiler_params=_PARALLEL,
    )(adj)


def _gcn_conv(x, adj_bf, dinv, w, b, *, relu, u=None, p_unit=None, w2=None,
              softmax_classes=None, n_tiles=2):
    M = adj_bf.shape[0]
    Cin = x.shape[1]
    Cout = w.shape[1]
    tile = M // n_tiles
    b2 = b.reshape(1, Cout).astype(jnp.float32)
    has_u = u is not None
    has_score = p_unit is not None
    has_y = w2 is not None

    inputs = [x.astype(jnp.float32), adj_bf, dinv,
              w.astype(jnp.bfloat16), b2]
    in_specs = [
        pl.BlockSpec((M, Cin), lambda i: (0, 0)),
        pl.BlockSpec((tile, M), lambda i: (i, 0)),
        pl.BlockSpec((M, 1), lambda i: (0, 0)),
        pl.BlockSpec((Cin, Cout), lambda i: (0, 0)),
        pl.BlockSpec((1, Cout), lambda i: (0, 0)),
    ]
    if has_u:
        inputs.append(u)
        in_specs.append(pl.BlockSpec((M, Cout), lambda i: (0, 0)))
    if has_score:
        inputs.append(p_unit.reshape(1, Cout).astype(jnp.float32))
        in_specs.append(pl.BlockSpec((1, Cout), lambda i: (0, 0)))
    if has_y:
        inputs.append(w2.astype(jnp.bfloat16))
        in_specs.append(
            pl.BlockSpec((w2.shape[0], w2.shape[1]), lambda i: (0, 0)))

    out_shape = [jax.ShapeDtypeStruct((M, Cout), jnp.float32)]
    out_specs = [pl.BlockSpec((tile, Cout), lambda i: (i, 0))]
    if has_score:
        out_shape.append(jax.ShapeDtypeStruct((M, 1), jnp.float32))
        out_specs.append(pl.BlockSpec((tile, 1), lambda i: (i, 0)))
    if has_y:
        C2 = w2.shape[1]
        out_shape.append(jax.ShapeDtypeStruct((M, C2), jnp.float32))
        out_specs.append(pl.BlockSpec((tile, C2), lambda i: (i, 0)))

    res = pl.pallas_call(
        partial(_gcn_body, relu=relu, has_u=has_u, has_score=has_score,
                has_y=has_y, softmax_classes=softmax_classes, tile=tile),
        out_shape=tuple(out_shape),
        grid=(n_tiles,),
        in_specs=in_specs,
        out_specs=tuple(out_specs),
        compiler_params=_PARALLEL,
    )(*inputs)
    return res if len(out_shape) > 1 else res[0]


def _bbt(bmat, gate, n_tiles):
    Mp, K = bmat.shape
    tile = Mp // n_tiles
    gcol = gate.reshape(Mp, 1)
    grow = gate.reshape(1, Mp)
    return pl.pallas_call(
        partial(_bbt_body, tile=tile),
        out_shape=(jax.ShapeDtypeStruct((Mp, Mp), jnp.int8),
                   jax.ShapeDtypeStruct((Mp, 1), jnp.float32)),
        grid=(n_tiles,),
        in_specs=[pl.BlockSpec((tile, K), lambda i: (i, 0)),
                  pl.BlockSpec((Mp, K), lambda i: (0, 0)),
                  pl.BlockSpec((tile, 1), lambda i: (i, 0)),
                  pl.BlockSpec((1, Mp), lambda i: (0, 0))],
        out_specs=(pl.BlockSpec((tile, Mp), lambda i: (i, 0)),
                   pl.BlockSpec((tile, 1), lambda i: (i, 0))),
        compiler_params=_PARALLEL,
    )(bmat, bmat, gcol, grow)


def _up_scatter(y, perm, k, rows):
    return jnp.zeros((rows, y.shape[1]), y.dtype).at[perm[:k]].set(y[:k])


def kernel(x_feat, pos, adj,
           down_w_0, down_w_1, down_w_2,
           down_b_0, down_b_1, down_b_2,
           pool_p_0, pool_p_1,
           up_w_0, up_w_1, up_b_0, up_b_1):
    N = adj.shape[0]
    num_classes = up_w_1.shape[1]
    c1 = up_w_0.shape[1]
    c0 = down_w_0.shape[1]

    p0_unit = pool_p_0 / jnp.linalg.norm(pool_p_0)
    p1_unit = pool_p_1 / jnp.linalg.norm(pool_p_1)

    w_up0_top = up_w_0[:c1]
    w_up0_bot = up_w_0[c1:]
    w_fin_pad = jnp.zeros((up_w_1.shape[0], 128),
                          jnp.float32).at[:, :num_classes].set(up_w_1)
    b_fin_pad = jnp.zeros((128,), jnp.float32).at[:num_classes].set(up_b_1)
    w_fin_top = w_fin_pad[:c0]
    w_fin_bot = w_fin_pad[c0:]

    asl0, dinv0 = _prep(adj.astype(jnp.float32))

    x0 = jnp.concatenate([x_feat, pos], axis=-1).astype(jnp.float32)
    x0out, score0 = _gcn_conv(x0, asl0, dinv0, down_w_0, down_b_0,
                              relu=True, p_unit=p0_unit)

    n0 = N
    k1 = int(math.ceil(0.8 * n0))
    kpad1 = min(_round_up(k1, 8), N)
    Mp1 = _round_up(kpad1, 128)
    perm1 = jnp.argsort(-score0[:, 0])[:kpad1]
    perm1 = jnp.concatenate(
        [perm1, jnp.zeros((Mp1 - kpad1,), perm1.dtype)])
    gate1 = (jnp.arange(Mp1) < k1).astype(jnp.float32)

    sc1 = score0[perm1, 0] * gate1
    x1 = x0out[perm1] * sc1[:, None]
    b1 = asl0[perm1]

    adj1, dinv1 = _bbt(b1, gate1.astype(jnp.int32), n_tiles=2)

    x1out, score1 = _gcn_conv(x1, adj1, dinv1, down_w_1, down_b_1,
                              relu=True, p_unit=p1_unit)

    n1 = k1
    k2 = int(math.ceil(0.8 * n1))
    kpad2 = min(_round_up(k2, 8), kpad1)
    Mp2 = _round_up(kpad2, 128)
    valid = jnp.arange(Mp1) < n1
    masked = jnp.where(valid, score1[:, 0], -jnp.inf)
    perm2 = jnp.argsort(-masked)[:kpad2]
    perm2 = jnp.concatenate(
        [perm2, jnp.zeros((Mp2 - kpad2,), perm2.dtype)])
    gate2 = (jnp.arange(Mp2) < k2).astype(jnp.float32)

    sc2 = score1[perm2, 0] * gate2
    x2 = x1out[perm2] * sc2[:, None]
    b2 = adj1[perm2]

    adj2, dinv2 = _bbt(b2, gate2.astype(jnp.int32), n_tiles=2)

    x2out, y2 = _gcn_conv(x2, adj2, dinv2, down_w_2, down_b_2,
                          relu=True, w2=w_up0_bot)

    u1 = _up_scatter(y2, perm2, k2, Mp1)
    xu1, y1 = _gcn_conv(x1out, adj1, dinv1, w_up0_top, up_b_0,
                        relu=True, u=u1, w2=w_fin_bot)

    u0 = _up_scatter(y1, perm1, k1, N)
    out = _gcn_conv(x0out, asl0, dinv0, w_fin_top, b_fin_pad, relu=False,
                    u=u0, softmax_classes=num_classes)
    return out[:, :num_classes]
